# Initial kernel scaffold; baseline (speedup 1.0000x reference)
import jax
import jax.numpy as jnp
from jax.experimental import pallas as pl


def kernel(pos, mag, cell):
    raise NotImplementedError("write your pallas kernel here")


# speedup vs baseline: 15.6288x; 15.6288x over previous
import numpy as np
import jax
import jax.numpy as jnp
from jax.experimental import pallas as pl

_CUTOFF = 4.5
_NR = 8
_EPS = 1e-08
_UREF = 2.2 * 2.2
_N = 128
_BA = 8

_centers_np = np.linspace(0.0, _CUTOFF, _NR).astype(np.float32)
_spacing = float(_CUTOFF / (_NR - 1))
_beta = float(1.0 / max(_spacing * _spacing, _EPS))
_SQRT2 = float(np.sqrt(2.0))

_ROW_MAGY = 10
_ROW_SY = 40
_ROW_UALL = 50
_ROW_DIAG = 51


def _desc_kernel(pos_blk_ref, mag_blk_ref, posT_ref, magT_ref, cell_ref,
                 icell_ref, rho_ref, arr_ref, amm_ref, aimm_ref):
    i0 = pl.program_id(0) * _BA
    posT = posT_ref[...]
    magT = magT_ref[...]
    cell = cell_ref[...]
    ic = icell_ref[...]
    pos_blk = pos_blk_ref[...]
    mag_blk = mag_blk_ref[...]

    def r16(x):
        return x.astype(jnp.bfloat16).astype(jnp.float32)

    disp = [posT[d][None, :] - pos_blk[:, d][:, None] for d in range(3)]
    disp = [r16(x) for x in disp]
    ic = r16(ic)
    cellr = r16(cell)
    frac = [disp[0] * ic[0, d] + disp[1] * ic[1, d] + disp[2] * ic[2, d]
            for d in range(3)]
    for _ in range(3):
        frac = [f - jnp.round(f) for f in frac]
    frac = [r16(x) for x in frac]
    rij = [frac[0] * cellr[0, d] + frac[1] * cellr[1, d] + frac[2] * cellr[2, d]
           for d in range(3)]
    sq = rij[0] * rij[0] + rij[1] * rij[1] + rij[2] * rij[2]
    pos_sq = sq > 0
    safe = jnp.where(pos_sq, sq, 1.0)
    inv_d = jnp.where(pos_sq, jax.lax.rsqrt(safe), 0.0)
    dist = sq * inv_d

    x = jnp.clip(np.float32(np.pi / _CUTOFF) * dist, 0.0, np.float32(np.pi))
    cut = 0.5 * (jnp.cos(x) + 1.0) * (dist < _CUTOFF).astype(jnp.float32)
    jidx = jax.lax.broadcasted_iota(jnp.int32, (_BA, _N), 1)
    iidx = i0 + jax.lax.broadcasted_iota(jnp.int32, (_BA, _N), 0)
    fm = cut * (jidx != iidx).astype(jnp.float32)

    centers = _spacing * jax.lax.broadcasted_iota(
        jnp.int32, (1, _NR, 1), 1).astype(jnp.float32)
    diff = dist[:, None, :] - centers
    f3 = jnp.exp(-_beta * diff * diff) * fm[:, None, :]

    u = [rij[d] * inv_d for d in range(3)]
    ones = jnp.ones((_BA, _N), jnp.float32)
    ylist = [ones, u[0], u[1], u[2],
             u[0] * u[0], _SQRT2 * u[0] * u[1], _SQRT2 * u[0] * u[2],
             u[1] * u[1], _SQRT2 * u[1] * u[2], u[2] * u[2]]
    Y = jnp.stack(ylist, axis=1)

    s = (magT[0][None, :] * mag_blk[:, 0][:, None]
         + magT[1][None, :] * mag_blk[:, 1][:, None]
         + magT[2][None, :] * mag_blk[:, 2][:, None])
    u_all = magT[0] * magT[0] + magT[1] * magT[1] + magT[2] * magT[2]
    u_row = jnp.broadcast_to(u_all[None, None, :], (_BA, 1, _N))

    c = sq / (sq + _EPS)
    pd2 = 1.5 * c * c - 0.5

    pieces = [Y]
    for d in range(3):
        pieces.append(magT[d][None, None, :] * Y)
    pieces.append(s[:, None, :] * Y)
    pieces.append(u_row)
    for w2 in (None, u_all[None, :], s * s):
        for pdl in (None, c, pd2):
            w = None
            if w2 is not None:
                w = w2
            if pdl is not None:
                w = pdl if w is None else w * pdl
            pieces.append(f3 if w is None else f3 * w[:, None, :]
                          if w.ndim == 2 else f3 * w[None, None, :])
    Z = jnp.concatenate(pieces, axis=1)

    M = jnp.stack(
        [jnp.sum(Z * f3[:, a, None, :], axis=-1) for a in range(_NR)],
        axis=-1)

    def gram(Xm):
        return jnp.sum(Xm[:, :, :, None] * Xm[:, :, None, :], axis=1)

    def triplet(base_rows, diag_base):
        S = M[:, base_rows, :]
        V = M[:, base_rows + 1:base_rows + 4, :]
        W = M[:, base_rows + 4:base_rows + 10, :]
        SS = S[:, :, None] * S[:, None, :]
        l0 = SS - M[:, diag_base:diag_base + 8, :]
        l1 = gram(V) - M[:, diag_base + 8:diag_base + 16, :]
        l2 = (1.5 * gram(W) - 0.5 * SS) - M[:, diag_base + 16:diag_base + 24, :]
        return jnp.stack([l0, l1, l2], axis=1)

    arr_ref[...] = triplet(0, _ROW_DIAG)

    Mg = M[:, _ROW_MAGY:_ROW_MAGY + 30, :].reshape(_BA, 3, 10, _NR)
    A = Mg[:, :, 0, :]
    B = Mg[:, :, 1:4, :].reshape(_BA, 9, _NR)
    C = Mg[:, :, 4:10, :].reshape(_BA, 18, _NR)
    AA = gram(A)
    db = _ROW_DIAG + 24
    mm0 = AA - M[:, db:db + 8, :]
    mm1 = gram(B) - M[:, db + 8:db + 16, :]
    mm2 = (1.5 * gram(C) - 0.5 * AA) - M[:, db + 16:db + 24, :]
    amm_ref[...] = jnp.stack([mm0, mm1, mm2], axis=1)

    aimm_ref[...] = triplet(_ROW_SY, _ROW_DIAG + 48)

    u_i = (mag_blk[:, 0] * mag_blk[:, 0] + mag_blk[:, 1] * mag_blk[:, 1]
           + mag_blk[:, 2] * mag_blk[:, 2])
    un = u_i / _UREF
    rho_u = jnp.stack([un, un * un, un * un * un], axis=1)
    rho_ref[...] = jnp.concatenate(
        [rho_u, M[:, 0, :], M[:, _ROW_UALL, :], M[:, _ROW_SY, :]], axis=1)


def _build(interpret=False):
    nb = _N // _BA
    return pl.pallas_call(
        _desc_kernel,
        grid=(nb,),
        in_specs=[
            pl.BlockSpec((_BA, 3), lambda b: (b, 0)),
            pl.BlockSpec((_BA, 3), lambda b: (b, 0)),
            pl.BlockSpec((3, _N), lambda b: (0, 0)),
            pl.BlockSpec((3, _N), lambda b: (0, 0)),
            pl.BlockSpec((3, 3), lambda b: (0, 0)),
            pl.BlockSpec((3, 3), lambda b: (0, 0)),
        ],
        out_specs=[
            pl.BlockSpec((_BA, 27), lambda b: (b, 0)),
            pl.BlockSpec((_BA, 3, _NR, _NR), lambda b: (b, 0, 0, 0)),
            pl.BlockSpec((_BA, 3, _NR, _NR), lambda b: (b, 0, 0, 0)),
            pl.BlockSpec((_BA, 3, _NR, _NR), lambda b: (b, 0, 0, 0)),
        ],
        out_shape=[
            jax.ShapeDtypeStruct((_N, 27), jnp.float32),
            jax.ShapeDtypeStruct((_N, 3, _NR, _NR), jnp.float32),
            jax.ShapeDtypeStruct((_N, 3, _NR, _NR), jnp.float32),
            jax.ShapeDtypeStruct((_N, 3, _NR, _NR), jnp.float32),
        ],
        interpret=interpret,
    )


def _run(pos, mag, cell, interpret=False):
    icell = jnp.linalg.inv(cell)
    rho, arr, amm, aimm = _build(interpret)(
        pos, mag, pos.T, mag.T, cell, icell)
    n = _N

    def flat(t):
        return t.transpose(0, 2, 3, 1).reshape(n, 3 * _NR * _NR)

    return jnp.concatenate(
        [rho[:, 0:3], rho[:, 3:11], flat(arr), rho[:, 11:19],
         rho[:, 19:27], flat(amm), flat(aimm)], axis=1)


def kernel(pos, mag, cell):
    return _run(pos, mag, cell, interpret=False)
